# baseline (speedup 1.0000x reference)
import functools

import jax
import jax.numpy as jnp
from jax import lax
from jax.experimental import pallas as pl
from jax.experimental.pallas import tpu as pltpu
from jax.experimental.pallas import tpu_sc as plsc

L = 16
CH = 128
EPS = 1e-12
STRIDES = (8, 4, 2, 1)


def _lane_of_token():
    import numpy as np
    lanes = np.arange(L)
    rng = np.random.RandomState(0)
    P = rng.rand(L, L)

    def comb(a, b, s):
        ared = a + a[lanes ^ s]
        bred = b + b[lanes ^ s]
        return np.where((lanes & s) == 0, ared, bred)

    acc = [None] * 5
    for m in range(L):
        p = P[m]
        lvl = 0
        while acc[lvl] is not None:
            p = comb(acc[lvl], p, STRIDES[lvl])
            acc[lvl] = None
            lvl += 1
        acc[lvl] = p
    totals = P.sum(axis=1)
    lane_of = [0] * L
    for lane in range(L):
        t = int(np.argmin(np.abs(totals - acc[4][lane])))
        assert abs(totals[t] - acc[4][lane]) < 1e-9
        lane_of[t] = lane
    return tuple(lane_of)


LANE_OF = _lane_of_token()


@functools.lru_cache(maxsize=4)
def _build(B, S, V, D):
    info = plsc.get_sparse_core_info()
    NC, NS = info.num_cores, info.num_subcores
    NW = NC * NS
    POS_PER_W = S // NW
    NCHUNK = B // CH
    DC = D // L
    NSTEP = POS_PER_W * NCHUNK
    NGBUF = 2
    NOBUF = 3

    mesh = plsc.VectorSubcoreMesh(core_axis_name="c", subcore_axis_name="s")

    @functools.partial(
        pl.kernel,
        mesh=mesh,
        out_type=jax.ShapeDtypeStruct((B, S, D), jnp.float32),
        scratch_types=[
            pltpu.VMEM((POS_PER_W, D), jnp.float32),
            pltpu.VMEM((NGBUF, CH, D), jnp.float32),
            pltpu.VMEM((NOBUF, CH, D), jnp.float32),
            pltpu.VMEM((POS_PER_W, B), jnp.int32),
            pltpu.VMEM((POS_PER_W, B), jnp.int32),
            pltpu.VMEM((D,), jnp.float32),
            pltpu.VMEM((D,), jnp.float32),
            pltpu.VMEM((D,), jnp.float32),
            pltpu.SemaphoreType.DMA,
            pltpu.SemaphoreType.DMA,
        ],
    )
    def emb_ln(ids_hbm, tt_hbm, word_hbm, pos_hbm, delta_hbm, gamma_hbm,
               beta_hbm, out_hbm, pos_v, wbuf, obuf, ids_v, tt_v, delta_v,
               gamma_v, beta_v, gsem, osem):
        wid = lax.axis_index("s") * NC + lax.axis_index("c")
        sbase = wid * POS_PER_W

        pltpu.sync_copy(pos_hbm.at[pl.ds(sbase, POS_PER_W)], pos_v)
        pltpu.sync_copy(delta_hbm, delta_v)
        pltpu.sync_copy(gamma_hbm, gamma_v)
        pltpu.sync_copy(beta_hbm, beta_v)
        pltpu.sync_copy(ids_hbm.at[pl.ds(sbase, POS_PER_W)], ids_v)
        pltpu.sync_copy(tt_hbm.at[pl.ds(sbase, POS_PER_W)], tt_v)

        dconst = [delta_v[pl.ds(k * L, L)] for k in range(DC)]
        gconst = [gamma_v[pl.ds(k * L, L)] for k in range(DC)]
        bconst = [beta_v[pl.ds(k * L, L)] for k in range(DC)]

        def tree_sum(vs):
            while len(vs) > 1:
                vs = [a + b for a, b in zip(vs[::2], vs[1::2])]
            return vs[0]

        lanes = jnp.arange(L, dtype=jnp.int32)

        def perm(x, idx):
            return lax.gather(
                x, idx[:, None],
                dimension_numbers=lax.GatherDimensionNumbers(
                    offset_dims=(), collapsed_slice_dims=(0,),
                    start_index_map=(0,)),
                slice_sizes=(1,),
                mode=lax.GatherScatterMode.PROMISE_IN_BOUNDS)

        def splat(x, m):
            return perm(x, jnp.full((L,), m, jnp.int32))

        MASKS = {s: (lanes & s) == 0 for s in STRIDES}

        def combine(a, b, s):
            ared = a + perm(a, lanes ^ s)
            bred = b + perm(b, lanes ^ s)
            return jnp.where(MASKS[s], ared, bred)

        def drain_gather():
            pltpu.make_async_copy(word_hbm.at[pl.ds(0, CH)], wbuf.at[0],
                                  gsem).wait()

        def drain_out():
            pltpu.make_async_copy(obuf.at[0],
                                  out_hbm.at[pl.ds(0, CH), 0], osem).wait()

        pltpu.async_copy(word_hbm.at[ids_v.at[0, pl.ds(0, CH)]], wbuf.at[0],
                         gsem)

        def step(gc, _):
            p = gc // NCHUNK
            cb = (gc % NCHUNK) * CH
            wi = gc % NGBUF
            oi = gc % NOBUF

            @pl.when(gc >= NOBUF)
            def _():
                drain_out()

            @pl.when(gc + 1 < NSTEP)
            def _():
                np_ = (gc + 1) // NCHUNK
                nb = ((gc + 1) % NCHUNK) * CH
                pltpu.async_copy(word_hbm.at[ids_v.at[np_, pl.ds(nb, CH)]],
                                 wbuf.at[(gc + 1) % NGBUF], gsem)

            drain_gather()

            posreg = [pos_v[p, pl.ds(k * L, L)] for k in range(DC)]

            def tok_load(j):
                return [wbuf[wi, j, pl.ds(k * L, L)] for k in range(DC)]

            @plsc.parallel_loop(0, CH // L)
            def grp_body(g):
                j0 = g * L
                tvec = tt_v[p, pl.ds(cb + j0, L)].astype(jnp.float32)

                acc1 = [None] * 5
                acc2 = [None] * 5
                cur = tok_load(j0)
                for m in range(L):
                    nxt = tok_load(j0 + m + 1) if m + 1 < L else None
                    t_f = splat(tvec, m)
                    w = [cur[k] + posreg[k] + t_f * dconst[k]
                         for k in range(DC)]
                    p1 = tree_sum(w)
                    p2 = tree_sum([x * x for x in w])
                    lvl = 0
                    while acc1[lvl] is not None:
                        s = STRIDES[lvl]
                        p1 = combine(acc1[lvl], p1, s)
                        p2 = combine(acc2[lvl], p2, s)
                        acc1[lvl] = None
                        acc2[lvl] = None
                        lvl += 1
                    acc1[lvl] = p1
                    acc2[lvl] = p2
                    cur = nxt
                meanv = acc1[4] * (1.0 / D)
                ex2v = acc2[4] * (1.0 / D)
                v = ex2v - meanv * meanv + EPS
                iv = lax.bitcast_convert_type(v, jnp.int32)
                y = lax.bitcast_convert_type(
                    jnp.int32(0x5F3759DF) - lax.shift_right_logical(iv, 1),
                    jnp.float32)
                for _ in range(2):
                    y = y * (1.5 - 0.5 * v * y * y)
                myv = meanv * y

                cur = tok_load(j0)
                for m in range(L):
                    j = j0 + m
                    nxt = tok_load(j + 1) if m + 1 < L else None
                    t_f = splat(tvec, m)
                    ysp = splat(y, LANE_OF[m])
                    mysp = splat(myv, LANE_OF[m])
                    for k in range(DC):
                        w = cur[k] + posreg[k] + t_f * dconst[k]
                        obuf[oi, j, pl.ds(k * L, L)] = (
                            (w * ysp - mysp) * gconst[k] + bconst[k])
                    cur = nxt

            pltpu.async_copy(obuf.at[oi], out_hbm.at[pl.ds(cb, CH), sbase + p],
                             osem)
            return 0

        lax.fori_loop(0, NSTEP, step, 0)
        for _ in range(NOBUF):
            drain_out()

    return emb_ln


def kernel(input_ids, token_type_ids, word_table, pos_table, type_table,
           gamma, beta):
    B, S = input_ids.shape
    V, D = word_table.shape
    posplus = pos_table + type_table[0][None, :]
    delta = type_table[1] - type_table[0]
    ids_t = input_ids.T
    tt_t = token_type_ids.T
    fn = _build(B, S, V, D)
    return fn(ids_t, tt_t, word_table, posplus, delta, gamma, beta)


# speedup vs baseline: 1.0623x; 1.0623x over previous
import functools

import jax
import jax.numpy as jnp
from jax import lax
from jax.experimental import pallas as pl
from jax.experimental.pallas import tpu as pltpu
from jax.experimental.pallas import tpu_sc as plsc

L = 16
CH = 128
EPS = 1e-12


@functools.lru_cache(maxsize=4)
def _build(B, S, V, D):
    info = plsc.get_sparse_core_info()
    NC, NS = info.num_cores, info.num_subcores
    NW = NC * NS
    POS_PER_W = S // NW
    NCHUNK = B // CH
    DC = D // L
    NSTEP = POS_PER_W * NCHUNK
    NGBUF = 2
    NOBUF = 3

    mesh = plsc.VectorSubcoreMesh(core_axis_name="c", subcore_axis_name="s")

    @functools.partial(
        pl.kernel,
        mesh=mesh,
        out_type=jax.ShapeDtypeStruct((B, S, D), jnp.float32),
        scratch_types=[
            pltpu.VMEM((POS_PER_W, D), jnp.float32),
            pltpu.VMEM((NGBUF, CH + 1, D), jnp.float32),
            pltpu.VMEM((NOBUF, CH, D), jnp.float32),
            pltpu.VMEM((POS_PER_W, B), jnp.int32),
            pltpu.VMEM((POS_PER_W, B), jnp.int32),
            pltpu.VMEM((D,), jnp.float32),
            pltpu.VMEM((D,), jnp.float32),
            pltpu.VMEM((D,), jnp.float32),
            pltpu.SemaphoreType.DMA,
            pltpu.SemaphoreType.DMA,
        ],
    )
    def emb_ln(ids_hbm, tt_hbm, word_hbm, pos_hbm, delta_hbm, gamma_hbm,
               beta_hbm, out_hbm, pos_v, wbuf, obuf, ids_v, tt_v, delta_v,
               gamma_v, beta_v, gsem, osem):
        wid = lax.axis_index("s") * NC + lax.axis_index("c")
        sbase = wid * POS_PER_W

        pltpu.sync_copy(pos_hbm.at[pl.ds(sbase, POS_PER_W)], pos_v)
        pltpu.sync_copy(delta_hbm, delta_v)
        pltpu.sync_copy(gamma_hbm, gamma_v)
        pltpu.sync_copy(beta_hbm, beta_v)
        pltpu.sync_copy(ids_hbm.at[pl.ds(sbase, POS_PER_W)], ids_v)
        pltpu.sync_copy(tt_hbm.at[pl.ds(sbase, POS_PER_W)], tt_v)

        dconst = [delta_v[pl.ds(k * L, L)] for k in range(DC)]
        gconst = [gamma_v[pl.ds(k * L, L)] for k in range(DC)]
        bconst = [beta_v[pl.ds(k * L, L)] for k in range(DC)]

        def tree_sum(vs):
            while len(vs) > 1:
                vs = [a + b for a, b in zip(vs[::2], vs[1::2])]
            return vs[0]

        lanes = jnp.arange(L, dtype=jnp.int32)

        def perm(x, idx):
            return lax.gather(
                x, idx[:, None],
                dimension_numbers=lax.GatherDimensionNumbers(
                    offset_dims=(), collapsed_slice_dims=(0,),
                    start_index_map=(0,)),
                slice_sizes=(1,),
                mode=lax.GatherScatterMode.PROMISE_IN_BOUNDS)

        def lane_sum(x):
            for m in (1, 2, 4, 8):
                x = x + perm(x, lanes ^ m)
            return x

        def splat(x, m):
            return perm(x, jnp.full((L,), m, jnp.int32))

        def drain_gather():
            pltpu.make_async_copy(word_hbm.at[pl.ds(0, CH)],
                                  wbuf.at[0, pl.ds(0, CH)], gsem).wait()

        def drain_out():
            pltpu.make_async_copy(obuf.at[0],
                                  out_hbm.at[pl.ds(0, CH), 0], osem).wait()

        pltpu.async_copy(word_hbm.at[ids_v.at[0, pl.ds(0, CH)]],
                         wbuf.at[0, pl.ds(0, CH)], gsem)

        def step(gc, _):
            p = gc // NCHUNK
            cb = (gc % NCHUNK) * CH
            wi = gc % NGBUF
            oi = gc % NOBUF

            @pl.when(gc >= NOBUF)
            def _():
                drain_out()

            @pl.when(gc + 1 < NSTEP)
            def _():
                np_ = (gc + 1) // NCHUNK
                nb = ((gc + 1) % NCHUNK) * CH
                pltpu.async_copy(word_hbm.at[ids_v.at[np_, pl.ds(nb, CH)]],
                                 wbuf.at[(gc + 1) % NGBUF, pl.ds(0, CH)],
                                 gsem)

            drain_gather()

            posreg = [pos_v[p, pl.ds(k * L, L)] for k in range(DC)]

            def tok_load(j):
                return [wbuf[wi, j, pl.ds(k * L, L)] for k in range(DC)]

            @plsc.parallel_loop(0, CH // L, carry=tuple(tok_load(0)))
            def grp_body(g, carry):
                j0 = g * L
                tvec = tt_v[p, pl.ds(cb + j0, L)].astype(jnp.float32)
                cur = list(carry)
                for m in range(L):
                    j = j0 + m
                    nxt = tok_load(j + 1)
                    t_f = splat(tvec, m)
                    w = [cur[k] + posreg[k] + t_f * dconst[k]
                         for k in range(DC)]
                    mean = lane_sum(tree_sum(w)) * (1.0 / D)
                    ex2 = lane_sum(tree_sum([x * x for x in w])) * (1.0 / D)
                    v = ex2 - mean * mean + EPS
                    iv = lax.bitcast_convert_type(v, jnp.int32)
                    y = lax.bitcast_convert_type(
                        jnp.int32(0x5F3759DF) - lax.shift_right_logical(iv, 1),
                        jnp.float32)
                    y = y * (1.5 - 0.5 * v * y * y)
                    for k in range(DC):
                        a = y * gconst[k]
                        obuf[oi, j, pl.ds(k * L, L)] = (
                            w[k] * a + (bconst[k] - mean * a))
                    cur = nxt
                return tuple(cur)

            pltpu.async_copy(obuf.at[oi], out_hbm.at[pl.ds(cb, CH), sbase + p],
                             osem)
            return 0

        lax.fori_loop(0, NSTEP, step, 0)
        for _ in range(NOBUF):
            drain_out()

    return emb_ln


def kernel(input_ids, token_type_ids, word_table, pos_table, type_table,
           gamma, beta):
    B, S = input_ids.shape
    V, D = word_table.shape
    posplus = pos_table + type_table[0][None, :]
    delta = type_table[1] - type_table[0]
    ids_t = input_ids.T
    tt_t = token_type_ids.T
    fn = _build(B, S, V, D)
    return fn(ids_t, tt_t, word_table, posplus, delta, gamma, beta)
